# baseline (speedup 1.0000x reference)
import math

import jax
import jax.numpy as jnp
from jax import lax
from jax.experimental import pallas as pl
from jax.experimental.pallas import tpu as pltpu
from jax.experimental.pallas import tpu_sc as plsc

D = 64
L = 16
NC, NS = 2, 16
NW = NC * NS
B = 16384 * 50
BPW = B // NW
C = 512
NCHUNK = BPW // C
SCALE = math.sqrt(D)


def _emb_body(table_hbm, tok_hbm, out_hbm,
              idx0, idx1, rows0, rows1, sem0, sem1):
    wid = lax.axis_index("s") * NC + lax.axis_index("c")
    base = wid * BPW
    idx = (idx0, idx1)
    rows = (rows0, rows1)
    sems = (sem0, sem1)

    for b in range(2):
        pltpu.sync_copy(tok_hbm.at[pl.ds(base + b * C, C)], idx[b])
        pltpu.async_copy(table_hbm.at[idx[b]], rows[b], sems[b])

    @pl.loop(0, NCHUNK, step=2)
    def _chunks(g):
        for b in range(2):
            gb = g + b
            pltpu.make_async_copy(
                table_hbm.at[idx[b]], rows[b], sems[b]).wait()

            @plsc.parallel_loop(0, C, 1, unroll=8)
            def _scale(r):
                for j in range(D // L):
                    sl = pl.ds(j * L, L)
                    rows[b][r, sl] = rows[b][r, sl] * SCALE

            pltpu.sync_copy(rows[b], out_hbm.at[pl.ds(base + gb * C, C)])

            @pl.when(gb + 2 < NCHUNK)
            def _fire():
                nxt = base + (gb + 2) * C
                pltpu.sync_copy(tok_hbm.at[pl.ds(nxt, C)], idx[b])
                pltpu.async_copy(table_hbm.at[idx[b]], rows[b], sems[b])


def kernel(tokens, table):
    tok_flat = tokens.reshape(-1)
    mesh = plsc.VectorSubcoreMesh(core_axis_name="c", subcore_axis_name="s")
    k = pl.kernel(
        _emb_body,
        out_type=jax.ShapeDtypeStruct((B, D), jnp.float32),
        mesh=mesh,
        scratch_types=[
            pltpu.VMEM((C,), jnp.int32),
            pltpu.VMEM((C,), jnp.int32),
            pltpu.VMEM((C, D), jnp.float32),
            pltpu.VMEM((C, D), jnp.float32),
            pltpu.SemaphoreType.DMA,
            pltpu.SemaphoreType.DMA,
        ],
        compiler_params=pltpu.CompilerParams(use_tc_tiling_on_sc=False),
    )
    out = k(table, tok_flat)
    return out.reshape(tokens.shape[0], tokens.shape[1], D)
